# baseline (speedup 1.0000x reference)
import functools

import jax
import jax.numpy as jnp
from jax import lax
from jax.experimental import pallas as pl
from jax.experimental.pallas import tpu as pltpu
from jax.experimental.pallas import tpu_sc as plsc

_R = 1024
_C = 100000
_K = 200
_NW = 32
_EPW = _R * _K // _NW
_CH = 128
_NCH = _EPW // _CH
_L = 16
_RPW = _R // _NW
_MAGIC = 335545
_SHIFT = 26


@functools.partial(
    pl.kernel,
    mesh=plsc.VectorSubcoreMesh(core_axis_name="c", subcore_axis_name="s"),
    out_type=jax.ShapeDtypeStruct((_NW, _NCH, _CH), jnp.float32),
    scratch_types=[
        pltpu.VMEM((_NCH, _CH), jnp.int32),
        pltpu.VMEM((_NCH, _CH), jnp.float32),
        pltpu.SemaphoreType.DMA,
    ],
)
def _sc_gather(tbl_hbm, idx_hbm, out_hbm, idx_v, out_v, sem):
    cid = lax.axis_index("c")
    sid = lax.axis_index("s")
    wid = sid * 2 + cid

    pltpu.sync_copy(idx_hbm.at[wid], idx_v)

    row0 = wid * _RPW

    def add_off(r, carry):
        for u in range(_CH // _L):
            sl = pl.ds(u * _L, _L)
            n = (r * _CH + u * _L) + lax.iota(jnp.int32, _L)
            row = row0 + ((n * _MAGIC) >> _SHIFT)
            idx_v[r, sl] = idx_v[r, sl] + row * _C
        return carry

    lax.fori_loop(0, _NCH, add_off, 0)

    def gath(g, carry):
        pltpu.async_copy(tbl_hbm.at[idx_v.at[g]], out_v.at[g], sem).wait()
        return carry

    lax.fori_loop(0, _NCH, gath, 0)

    pltpu.sync_copy(out_v, out_hbm.at[wid])


def kernel(input_tensor, indices):
    tbl = input_tensor.reshape(-1)
    idx3d = indices.reshape(_NW, _NCH, _CH)
    out = _sc_gather(tbl, idx3d)
    return out.reshape(_R, _K)


# speedup vs baseline: 26.3318x; 26.3318x over previous
import functools

import jax
import jax.numpy as jnp
from jax import lax
from jax.experimental import pallas as pl
from jax.experimental.pallas import tpu as pltpu
from jax.experimental.pallas import tpu_sc as plsc

_R = 1024
_C = 100000
_K = 200
_NW = 32
_EPW = _R * _K // _NW
_CH = 128
_NCH = _EPW // _CH
_L = 16
_RPW = _R // _NW
_MAGIC = 335545
_SHIFT = 26


@functools.partial(
    pl.kernel,
    mesh=plsc.VectorSubcoreMesh(core_axis_name="c", subcore_axis_name="s"),
    out_type=jax.ShapeDtypeStruct((_NW, _NCH, _CH), jnp.float32),
    scratch_types=[
        pltpu.VMEM((_NCH, _CH), jnp.int32),
        pltpu.VMEM((_NCH, _CH), jnp.float32),
        pltpu.SemaphoreType.DMA,
    ],
)
def _sc_gather(tbl_hbm, idx_hbm, out_hbm, idx_v, out_v, sem):
    cid = lax.axis_index("c")
    sid = lax.axis_index("s")
    wid = sid * 2 + cid

    pltpu.sync_copy(idx_hbm.at[wid], idx_v)

    row0 = wid * _RPW

    def add_off(r, carry):
        for u in range(_CH // _L):
            sl = pl.ds(u * _L, _L)
            n = (r * _CH + u * _L) + lax.iota(jnp.int32, _L)
            i = row0 + ((n * _MAGIC) >> _SHIFT)
            q = idx_v[r, sl]
            p = (((q >> 3) << 13) + ((i >> 7) << 10)
                 + ((q & 7) << 7) + (i & 127))
            idx_v[r, sl] = p
        return carry

    lax.fori_loop(0, _NCH, add_off, 0)

    def fire(g, carry):
        pltpu.async_copy(tbl_hbm.at[idx_v.at[g]], out_v.at[g], sem)
        return carry

    lax.fori_loop(0, _NCH, fire, 0)

    def drain(g, carry):
        pltpu.make_async_copy(tbl_hbm.at[idx_v.at[g]], out_v.at[g], sem).wait()
        return carry

    lax.fori_loop(0, _NCH, drain, 0)

    pltpu.sync_copy(out_v, out_hbm.at[wid])


def kernel(input_tensor, indices):
    tbl = (input_tensor.T.reshape(_C // 8, 8, _R // 128, 128)
           .transpose(0, 2, 1, 3).reshape(-1))
    idx3d = indices.reshape(_NW, _NCH, _CH)
    out = _sc_gather(tbl, idx3d)
    return out.reshape(_R, _K)
